# Initial kernel scaffold; baseline (speedup 1.0000x reference)
import jax
import jax.numpy as jnp
from jax.experimental import pallas as pl


def kernel(x, edge_index, Wl1, bl1, Wr1, Wl2, bl2, Wr2, Wl3, bl3, Wr3, fc1_w, fc1_b, fc2_w, fc2_b):
    raise NotImplementedError("write your pallas kernel here")


# speedup vs baseline: 10.6977x; 10.6977x over previous
import functools

import jax
import jax.numpy as jnp
from jax import lax
from jax.experimental import pallas as pl
from jax.experimental.pallas import tpu as pltpu
from jax.experimental.pallas import tpu_sc as plsc

N_NODES = 100000
N_EDGES = 1600000
BLK = 128
NBLK = N_EDGES // BLK
SLAB = 16
NSLAB = NBLK // SLAB
TAIL = NBLK - NSLAB * SLAB
NC = 2
NS = 16
NW = NC * NS
CHUNK = 6256
LAST_CHUNK = N_NODES - (NS - 1) * CHUNK
D = 16

_mesh = plsc.VectorSubcoreMesh(core_axis_name="c", subcore_axis_name="s")

_seg_scratch = [
    pltpu.VMEM((SLAB, BLK), jnp.int32),
    pltpu.VMEM((SLAB, BLK), jnp.int32),
    pltpu.VMEM((BLK, D), jnp.float32),
    pltpu.VMEM_SHARED((N_NODES, D), jnp.float32),
    pltpu.SemaphoreType.DMA,
]


def _do_block(table, acc, srcb, dstb, rows, sem, j):
    pltpu.async_copy(table.at[srcb.at[j]], rows, sem).wait()
    pltpu.sync_copy(rows, acc.at[dstb.at[j]], add=True)


def _edge_loop(src, dst, table, acc, srcb, dstb, rows, sem, first, stride,
               do_tail):
    nsl = (NSLAB - first + stride - 1) // stride

    def step(i, carry):
        slab = first + i * stride
        pltpu.sync_copy(src.at[pl.ds(slab * SLAB, SLAB)], srcb)
        pltpu.sync_copy(dst.at[pl.ds(slab * SLAB, SLAB)], dstb)
        for j in range(SLAB):
            _do_block(table, acc, srcb, dstb, rows, sem, j)
        return carry

    lax.fori_loop(0, nsl, step, 0)

    @pl.when(do_tail)
    def _():
        pltpu.sync_copy(src.at[pl.ds(NSLAB * SLAB, TAIL)],
                        srcb.at[pl.ds(0, TAIL)])
        pltpu.sync_copy(dst.at[pl.ds(NSLAB * SLAB, TAIL)],
                        dstb.at[pl.ds(0, TAIL)])
        for j in range(TAIL):
            _do_block(table, acc, srcb, dstb, rows, sem, j)


def _zero_acc(zt, acc, s):
    row0 = s * CHUNK

    @pl.when(s < NS - 1)
    def _():
        pltpu.sync_copy(zt, acc.at[pl.ds(row0, CHUNK)])

    @pl.when(s == NS - 1)
    def _():
        pltpu.sync_copy(zt.at[pl.ds(0, LAST_CHUNK)],
                        acc.at[pl.ds(row0, LAST_CHUNK)])


def _writeback(acc, out, c, s):
    row0 = s * CHUNK

    @pl.when(s < NS - 1)
    def _():
        pltpu.sync_copy(acc.at[pl.ds(row0, CHUNK)],
                        out.at[c, pl.ds(row0, CHUNK)])

    @pl.when(s == NS - 1)
    def _():
        pltpu.sync_copy(acc.at[pl.ds(row0, LAST_CHUNK)],
                        out.at[c, pl.ds(row0, LAST_CHUNK)])


@functools.partial(
    pl.kernel,
    mesh=_mesh,
    out_type=jax.ShapeDtypeStruct((NC, N_NODES, D), jnp.float32),
    scratch_types=_seg_scratch,
    compiler_params=pltpu.CompilerParams(use_tc_tiling_on_sc=False),
)
def _seg_p(src, dst, table, zt, out, srcb, dstb, rows, acc, sem):
    c = lax.axis_index("c")
    s = lax.axis_index("s")
    wid = s * NC + c
    _zero_acc(zt, acc, s)
    plsc.subcore_barrier()
    _edge_loop(src, dst, table, acc, srcb, dstb, rows, sem, wid, NW, wid == 0)
    plsc.subcore_barrier()
    _writeback(acc, out, c, s)


@functools.partial(
    pl.kernel,
    mesh=_mesh,
    out_type=jax.ShapeDtypeStruct((NC, N_NODES, D), jnp.float32),
    scratch_types=_seg_scratch,
    compiler_params=pltpu.CompilerParams(use_tc_tiling_on_sc=False),
)
def _seg_f(src, dst, ta, tb, zt, out, srcb, dstb, rows, acc, sem):
    c = lax.axis_index("c")
    s = lax.axis_index("s")
    _zero_acc(zt, acc, s)
    plsc.subcore_barrier()

    @pl.when(c == 0)
    def _():
        _edge_loop(src, dst, ta, acc, srcb, dstb, rows, sem, s, NS, s == 0)

    @pl.when(c == 1)
    def _():
        _edge_loop(src, dst, tb, acc, srcb, dstb, rows, sem, s, NS, s == 0)

    plsc.subcore_barrier()
    _writeback(acc, out, c, s)


BR = 2000


def _dense1_body(p1, x, wl1, bl1, wr1, wl2, h1_o, p2a_o, p2b_o, inv_o):
    agg = p1[0] + p1[1]
    inv = 1.0 / jnp.maximum(agg[:, 8:9], 1.0)
    mean1 = agg[:, 0:8] * inv
    h1 = jnp.maximum(mean1 @ wl1[...].T + bl1[...] + x[...] @ wr1[...].T, 0.0)
    h1_o[...] = h1
    p2a_o[...] = h1 @ wl2[...][0:16].T
    p2b_o[...] = h1 @ wl2[...][16:32].T
    inv_o[...] = inv


def _dense2_body(agg2, h1, inv, wr2, bl2, wl3, h2_o, p3_o):
    mean2 = jnp.concatenate([agg2[0], agg2[1]], axis=1) * inv[...]
    h2 = jnp.maximum(mean2 + bl2[...] + h1[...] @ wr2[...].T, 0.0)
    h2_o[...] = h2
    p3_o[...] = h2 @ wl3[...].T


def _dense3_body(p3, h2, inv, wr3, bl3, f1w, f1b, f2w, f2b, out_o):
    mean3 = (p3[0] + p3[1]) * inv[...]
    h3 = jnp.maximum(mean3 + bl3[...] + h2[...] @ wr3[...].T, 0.0)
    h4 = jnp.maximum(h3 @ f1w[...].T + f1b[...], 0.0)
    logits = h4 @ f2w[...].T + f2b[...]
    z = logits - jnp.max(logits, axis=1, keepdims=True)
    out_o[...] = z - jnp.log(jnp.sum(jnp.exp(z), axis=1, keepdims=True))


def _rows(d):
    return pl.BlockSpec((BR, d), lambda i: (i, 0))


def _part(d):
    return pl.BlockSpec((NC, BR, d), lambda i: (0, i, 0))


def _full(shape):
    return pl.BlockSpec(shape, lambda i: tuple(0 for _ in shape))


_G = N_NODES // BR


def kernel(x, edge_index, Wl1, bl1, Wr1, Wl2, bl2, Wr2, Wl3, bl3, Wr3,
           fc1_w, fc1_b, fc2_w, fc2_b):
    f32 = jnp.float32
    src2 = edge_index[0].reshape(NBLK, BLK)
    dst2 = edge_index[1].reshape(NBLK, BLK)
    zt = jnp.zeros((CHUNK, D), f32)

    table1 = jnp.concatenate(
        [x, jnp.ones((N_NODES, 1), f32), jnp.zeros((N_NODES, 7), f32)], axis=1)
    part1 = _seg_p(src2, dst2, table1, zt)

    h1, p2a, p2b, inv = pl.pallas_call(
        _dense1_body,
        grid=(_G,),
        in_specs=[_part(D), _rows(8), _full((64, 8)), _full((1, 64)),
                  _full((64, 8)), _full((32, 64))],
        out_specs=[_rows(64), _rows(16), _rows(16), _rows(1)],
        out_shape=[jax.ShapeDtypeStruct((N_NODES, 64), f32),
                   jax.ShapeDtypeStruct((N_NODES, 16), f32),
                   jax.ShapeDtypeStruct((N_NODES, 16), f32),
                   jax.ShapeDtypeStruct((N_NODES, 1), f32)],
    )(part1, x, Wl1, bl1.reshape(1, 64), Wr1, Wl2)

    agg2 = _seg_f(src2, dst2, p2a, p2b, zt)

    h2, p3 = pl.pallas_call(
        _dense2_body,
        grid=(_G,),
        in_specs=[_part(D), _rows(64), _rows(1), _full((32, 64)),
                  _full((1, 32)), _full((16, 32))],
        out_specs=[_rows(32), _rows(16)],
        out_shape=[jax.ShapeDtypeStruct((N_NODES, 32), f32),
                   jax.ShapeDtypeStruct((N_NODES, 16), f32)],
    )(agg2, h1, inv, Wr2, bl2.reshape(1, 32), Wl3)

    part3 = _seg_p(src2, dst2, p3, zt)

    out = pl.pallas_call(
        _dense3_body,
        grid=(_G,),
        in_specs=[_part(D), _rows(32), _rows(1), _full((16, 32)),
                  _full((1, 16)), _full((8, 16)), _full((1, 8)),
                  _full((2, 8)), _full((1, 2))],
        out_specs=_rows(2),
        out_shape=jax.ShapeDtypeStruct((N_NODES, 2), f32),
    )(part3, h2, inv, Wr3, bl3.reshape(1, 16), fc1_w, fc1_b.reshape(1, 8),
      fc2_w, fc2_b.reshape(1, 2))

    return out


# speedup vs baseline: 17.6294x; 1.6480x over previous
import functools

import jax
import jax.numpy as jnp
from jax import lax
from jax.experimental import pallas as pl
from jax.experimental.pallas import tpu as pltpu
from jax.experimental.pallas import tpu_sc as plsc

N_NODES = 100000
N_EDGES = 1600000
BLK = 128
NBLK = N_EDGES // BLK
SLAB = 16
NSLAB = NBLK // SLAB
TAIL = NBLK - NSLAB * SLAB
NC = 2
NS = 16
NW = NC * NS
CHUNK = 6256
LAST_CHUNK = N_NODES - (NS - 1) * CHUNK
D = 16

_mesh = plsc.VectorSubcoreMesh(core_axis_name="c", subcore_axis_name="s")

NB = 4
LA = 2

_seg_scratch = [
    pltpu.VMEM((SLAB, BLK), jnp.int32),
    pltpu.VMEM((SLAB, BLK), jnp.int32),
    pltpu.VMEM((NB, BLK, D), jnp.float32),
    pltpu.VMEM_SHARED((N_NODES, D), jnp.float32),
    pltpu.SemaphoreType.DMA((NB,)),
    pltpu.SemaphoreType.DMA((NB,)),
]


def _edge_loop(src, dst, table, acc, srcb, dstb, rows, gsem, ssem,
               first, stride, do_tail):
    nsl = (NSLAB - first + stride - 1) // stride

    def run_slab(nblocks):
        g = [None] * nblocks
        sc = [None] * nblocks
        for j in range(min(LA, nblocks)):
            g[j] = pltpu.async_copy(
                table.at[srcb.at[j]], rows.at[j % NB], gsem.at[j % NB])
        for j in range(nblocks):
            if j + LA < nblocks:
                b2 = (j + LA) % NB
                if j + LA >= NB:
                    sc[j + LA - NB].wait()
                g[j + LA] = pltpu.async_copy(
                    table.at[srcb.at[j + LA]], rows.at[b2], gsem.at[b2])
            g[j].wait()
            sc[j] = pltpu.async_copy(
                rows.at[j % NB], acc.at[dstb.at[j]], ssem.at[j % NB],
                add=True)
        for j in range(max(0, nblocks - NB), nblocks):
            sc[j].wait()

    def step(i, carry):
        slab = first + i * stride
        pltpu.sync_copy(src.at[pl.ds(slab * SLAB, SLAB)], srcb)
        pltpu.sync_copy(dst.at[pl.ds(slab * SLAB, SLAB)], dstb)
        run_slab(SLAB)
        return carry

    lax.fori_loop(0, nsl, step, 0)

    @pl.when(do_tail)
    def _():
        pltpu.sync_copy(src.at[pl.ds(NSLAB * SLAB, TAIL)],
                        srcb.at[pl.ds(0, TAIL)])
        pltpu.sync_copy(dst.at[pl.ds(NSLAB * SLAB, TAIL)],
                        dstb.at[pl.ds(0, TAIL)])
        run_slab(TAIL)


def _zero_acc(zt, acc, s):
    row0 = s * CHUNK

    @pl.when(s < NS - 1)
    def _():
        pltpu.sync_copy(zt, acc.at[pl.ds(row0, CHUNK)])

    @pl.when(s == NS - 1)
    def _():
        pltpu.sync_copy(zt.at[pl.ds(0, LAST_CHUNK)],
                        acc.at[pl.ds(row0, LAST_CHUNK)])


def _writeback(acc, out, c, s):
    row0 = s * CHUNK

    @pl.when(s < NS - 1)
    def _():
        pltpu.sync_copy(acc.at[pl.ds(row0, CHUNK)],
                        out.at[c, pl.ds(row0, CHUNK)])

    @pl.when(s == NS - 1)
    def _():
        pltpu.sync_copy(acc.at[pl.ds(row0, LAST_CHUNK)],
                        out.at[c, pl.ds(row0, LAST_CHUNK)])


@functools.partial(
    pl.kernel,
    mesh=_mesh,
    out_type=jax.ShapeDtypeStruct((NC, N_NODES, D), jnp.float32),
    scratch_types=_seg_scratch,
    compiler_params=pltpu.CompilerParams(use_tc_tiling_on_sc=False),
)
def _seg_p(src, dst, table, zt, out, srcb, dstb, rows, acc, gsem, ssem):
    c = lax.axis_index("c")
    s = lax.axis_index("s")
    wid = s * NC + c
    _zero_acc(zt, acc, s)
    plsc.subcore_barrier()
    _edge_loop(src, dst, table, acc, srcb, dstb, rows, gsem, ssem, wid, NW,
               wid == 0)
    plsc.subcore_barrier()
    _writeback(acc, out, c, s)


@functools.partial(
    pl.kernel,
    mesh=_mesh,
    out_type=jax.ShapeDtypeStruct((NC, N_NODES, D), jnp.float32),
    scratch_types=_seg_scratch,
    compiler_params=pltpu.CompilerParams(use_tc_tiling_on_sc=False),
)
def _seg_f(src, dst, ta, tb, zt, out, srcb, dstb, rows, acc, gsem, ssem):
    c = lax.axis_index("c")
    s = lax.axis_index("s")
    _zero_acc(zt, acc, s)
    plsc.subcore_barrier()

    @pl.when(c == 0)
    def _():
        _edge_loop(src, dst, ta, acc, srcb, dstb, rows, gsem, ssem, s, NS, s == 0)

    @pl.when(c == 1)
    def _():
        _edge_loop(src, dst, tb, acc, srcb, dstb, rows, gsem, ssem, s, NS, s == 0)

    plsc.subcore_barrier()
    _writeback(acc, out, c, s)


BR = 2000


def _dense1_body(p1, x, wl1, bl1, wr1, wl2, h1_o, p2a_o, p2b_o, inv_o):
    agg = p1[0] + p1[1]
    inv = 1.0 / jnp.maximum(agg[:, 8:9], 1.0)
    mean1 = agg[:, 0:8] * inv
    h1 = jnp.maximum(mean1 @ wl1[...].T + bl1[...] + x[...] @ wr1[...].T, 0.0)
    h1_o[...] = h1
    p2a_o[...] = h1 @ wl2[...][0:16].T
    p2b_o[...] = h1 @ wl2[...][16:32].T
    inv_o[...] = inv


def _dense2_body(agg2, h1, inv, wr2, bl2, wl3, h2_o, p3_o):
    mean2 = jnp.concatenate([agg2[0], agg2[1]], axis=1) * inv[...]
    h2 = jnp.maximum(mean2 + bl2[...] + h1[...] @ wr2[...].T, 0.0)
    h2_o[...] = h2
    p3_o[...] = h2 @ wl3[...].T


def _dense3_body(p3, h2, inv, wr3, bl3, f1w, f1b, f2w, f2b, out_o):
    mean3 = (p3[0] + p3[1]) * inv[...]
    h3 = jnp.maximum(mean3 + bl3[...] + h2[...] @ wr3[...].T, 0.0)
    h4 = jnp.maximum(h3 @ f1w[...].T + f1b[...], 0.0)
    logits = h4 @ f2w[...].T + f2b[...]
    z = logits - jnp.max(logits, axis=1, keepdims=True)
    out_o[...] = z - jnp.log(jnp.sum(jnp.exp(z), axis=1, keepdims=True))


def _rows(d):
    return pl.BlockSpec((BR, d), lambda i: (i, 0))


def _part(d):
    return pl.BlockSpec((NC, BR, d), lambda i: (0, i, 0))


def _full(shape):
    return pl.BlockSpec(shape, lambda i: tuple(0 for _ in shape))


_G = N_NODES // BR


def kernel(x, edge_index, Wl1, bl1, Wr1, Wl2, bl2, Wr2, Wl3, bl3, Wr3,
           fc1_w, fc1_b, fc2_w, fc2_b):
    f32 = jnp.float32
    src2 = edge_index[0].reshape(NBLK, BLK)
    dst2 = edge_index[1].reshape(NBLK, BLK)
    zt = jnp.zeros((CHUNK, D), f32)

    table1 = jnp.concatenate(
        [x, jnp.ones((N_NODES, 1), f32), jnp.zeros((N_NODES, 7), f32)], axis=1)
    part1 = _seg_p(src2, dst2, table1, zt)

    h1, p2a, p2b, inv = pl.pallas_call(
        _dense1_body,
        grid=(_G,),
        in_specs=[_part(D), _rows(8), _full((64, 8)), _full((1, 64)),
                  _full((64, 8)), _full((32, 64))],
        out_specs=[_rows(64), _rows(16), _rows(16), _rows(1)],
        out_shape=[jax.ShapeDtypeStruct((N_NODES, 64), f32),
                   jax.ShapeDtypeStruct((N_NODES, 16), f32),
                   jax.ShapeDtypeStruct((N_NODES, 16), f32),
                   jax.ShapeDtypeStruct((N_NODES, 1), f32)],
    )(part1, x, Wl1, bl1.reshape(1, 64), Wr1, Wl2)

    agg2 = _seg_f(src2, dst2, p2a, p2b, zt)

    h2, p3 = pl.pallas_call(
        _dense2_body,
        grid=(_G,),
        in_specs=[_part(D), _rows(64), _rows(1), _full((32, 64)),
                  _full((1, 32)), _full((16, 32))],
        out_specs=[_rows(32), _rows(16)],
        out_shape=[jax.ShapeDtypeStruct((N_NODES, 32), f32),
                   jax.ShapeDtypeStruct((N_NODES, 16), f32)],
    )(agg2, h1, inv, Wr2, bl2.reshape(1, 32), Wl3)

    part3 = _seg_p(src2, dst2, p3, zt)

    out = pl.pallas_call(
        _dense3_body,
        grid=(_G,),
        in_specs=[_part(D), _rows(32), _rows(1), _full((16, 32)),
                  _full((1, 16)), _full((8, 16)), _full((1, 8)),
                  _full((2, 8)), _full((1, 2))],
        out_specs=_rows(2),
        out_shape=jax.ShapeDtypeStruct((N_NODES, 2), f32),
    )(part3, h2, inv, Wr3, bl3.reshape(1, 16), fc1_w, fc1_b.reshape(1, 8),
      fc2_w, fc2_b.reshape(1, 2))

    return out
